# baseline (speedup 1.0000x reference)
import functools

import jax
import jax.numpy as jnp
from jax import lax
from jax.experimental import pallas as pl
from jax.experimental.pallas import tpu as pltpu
from jax.experimental.pallas import tpu_sc as plsc

N = 10000
E = 160000
D = 256
H = 256
C = 40
C_PAD = 128
HALF = 128

NC = 2
NS = 16
CHUNK = 128
NCHUNK = 1280
E_PAD = NCHUNK * CHUNK
CPT1 = NCHUNK // NS
CPT2 = NCHUNK // (NC * NS)
NBUF = 4
R0 = 624
RL = N - (NS - 1) * R0

BLK = 400
GRID = N // BLK

_mesh = plsc.VectorSubcoreMesh(
    core_axis_name="c", subcore_axis_name="s", num_cores=NC, num_subcores=NS
)


def _zero_fill(zero_hbm, acc, s):
    @pl.when(s < NS - 1)
    def _():
        pltpu.sync_copy(zero_hbm.at[pl.ds(0, R0)], acc.at[pl.ds(s * R0, R0)])

    @pl.when(s == NS - 1)
    def _():
        pltpu.sync_copy(zero_hbm, acc.at[pl.ds((NS - 1) * R0, RL)])


def _writeback(acc, out_hbm, c, s):
    @pl.when(s < NS - 1)
    def _():
        pltpu.sync_copy(acc.at[pl.ds(s * R0, R0)],
                        out_hbm.at[c, pl.ds(s * R0, R0)])

    @pl.when(s == NS - 1)
    def _():
        pltpu.sync_copy(acc.at[pl.ds((NS - 1) * R0, RL)],
                        out_hbm.at[c, pl.ds((NS - 1) * R0, RL)])

@functools.partial(
    pl.kernel,
    out_type=jax.ShapeDtypeStruct((NC, N, 128), jnp.float32),
    mesh=_mesh,
    scratch_types=[
        pltpu.VMEM((CPT2, CHUNK), jnp.int32),
        pltpu.VMEM((CHUNK, 128), jnp.float32),
        pltpu.VMEM_SHARED((N + 8, 128), jnp.float32),
    ] + [pltpu.SemaphoreType.DMA] * NBUF,
)
def _sc_degree(dst_hbm, ones_hbm, zero_hbm, out_hbm, idx_v, ones_v, acc,
               s0, s1, s2, s3):
    c = lax.axis_index("c")
    s = lax.axis_index("s")
    ssem = (s0, s1, s2, s3)
    base = c * (NCHUNK // NC) + s * CPT2
    pltpu.sync_copy(dst_hbm.at[pl.ds(base, CPT2)], idx_v)
    pltpu.sync_copy(ones_hbm, ones_v)
    _zero_fill(zero_hbm, acc, s)
    plsc.subcore_barrier()

    for b in range(NBUF):
        pltpu.async_copy(ones_v, acc.at[idx_v.at[b]], ssem[b], add=True)

    def body(g, carry):
        for b in range(NBUF):
            _drain_scatter(ones_v, acc, idx_v.at[0], ssem[b])
            pltpu.async_copy(ones_v, acc.at[idx_v.at[g * NBUF + b]],
                             ssem[b], add=True)
        return carry

    lax.fori_loop(1, CPT2 // NBUF, body, 0)
    for b in range(NBUF):
        _drain_scatter(ones_v, acc, idx_v.at[0], ssem[b])
    plsc.subcore_barrier()
    _writeback(acc, out_hbm, c, s)


def _drain_gather(table_hbm, rows_v, b, sem):
    pltpu.make_async_copy(table_hbm.at[pl.ds(0, CHUNK)], rows_v.at[b],
                          sem).wait()


def _drain_scatter(ones_v, acc, idx_row, sem):
    pltpu.make_async_copy(ones_v, acc.at[idx_row], sem).wait()


def _make_prop(split_edges):
    cpt = CPT2 if split_edges else CPT1

    @functools.partial(
        pl.kernel,
        out_type=jax.ShapeDtypeStruct((NC, N, 128), jnp.float32),
        mesh=_mesh,
        scratch_types=[
            pltpu.VMEM((4, 2, CHUNK), jnp.int32),
            pltpu.VMEM((2, CHUNK, 128), jnp.float32),
            pltpu.VMEM_SHARED((N + 8, 128), jnp.float32),
        ] + [pltpu.SemaphoreType.DMA] * 8,
    )
    def prop(tbl_hbm, idx_hbm, zero_hbm, out_hbm,
             idx_v, rows_v, acc, g0, g1, t0, t1, i0, i1, i2, i3):
        c = lax.axis_index("c")
        s = lax.axis_index("s")
        gsem = (g0, g1)
        ssem = (t0, t1)
        isem = (i0, i1, i2, i3)
        if split_edges:
            base = c * (NCHUNK // NC) + s * cpt
        else:
            base = s * cpt

        def issue_idx(j, q):
            pltpu.async_copy(idx_hbm.at[base + j], idx_v.at[q], isem[q])

        def wait_idx(q):
            pltpu.make_async_copy(idx_hbm.at[0], idx_v.at[q], isem[q]).wait()

        def issue_gather(q, b):
            if not split_edges:
                @pl.when(c > 0)
                def _():
                    for tt in range(CHUNK // 16):
                        sl = pl.ds(tt * 16, 16)
                        idx_v[q, 0, sl] = idx_v[q, 0, sl] + N
            pltpu.async_copy(tbl_hbm.at[idx_v.at[q, 0]], rows_v.at[b],
                             gsem[b])

        def wait_gather(b):
            pltpu.make_async_copy(tbl_hbm.at[idx_v.at[0, 0]], rows_v.at[b],
                                  gsem[b]).wait()

        def issue_scatter(q, b):
            pltpu.async_copy(rows_v.at[b], acc.at[idx_v.at[q, 1]], ssem[b],
                             add=True)

        def wait_scatter(b):
            pltpu.make_async_copy(rows_v.at[b], acc.at[idx_v.at[0, 1]],
                                  ssem[b]).wait()

        for q in range(4):
            issue_idx(q, q)
        _zero_fill(zero_hbm, acc, s)
        plsc.subcore_barrier()
        for b in range(2):
            wait_idx(b)
            issue_gather(b, b)

        def body(g, carry):
            j0 = g * 4
            for u in range(4):
                j = j0 + u
                b = u % 2
                wait_gather(b)
                issue_scatter(u, b)
                wait_scatter(b)
                issue_idx(j + 4, u)
                wait_idx((u + 2) % 4)
                issue_gather((u + 2) % 4, b)
            return carry

        lax.fori_loop(0, cpt // 4 - 1, body, 0)
        for u in range(4):
            b = u % 2
            wait_gather(b)
            issue_scatter(u, b)
            wait_scatter(b)
            if u < 2:
                wait_idx((u + 2) % 4)
                issue_gather((u + 2) % 4, b)
        plsc.subcore_barrier()
        _writeback(acc, out_hbm, c, s)

    return prop


_sc_prop1 = _make_prop(split_edges=False)
_sc_prop2 = _make_prop(split_edges=True)


def _dinv_body(degp_ref, dinv_ref):
    deg = degp_ref[0, :, 0] + degp_ref[1, :, 0] + 1.0
    dinv_ref[:, 0] = 1.0 / jnp.sqrt(deg)


def _tc_dinv(degp):
    return pl.pallas_call(
        _dinv_body,
        out_shape=jax.ShapeDtypeStruct((N, 1), jnp.float32),
    )(degp)


def _l1_body(x_ref, w1_ref, dinv_ref, hp_ref):
    h = jnp.dot(x_ref[...], w1_ref[...], preferred_element_type=jnp.float32)
    hp = dinv_ref[...] * h
    hp_ref[0] = hp[:, :HALF]
    hp_ref[1] = hp[:, HALF:]


def _tc_l1(x, w1, dinv):
    return pl.pallas_call(
        _l1_body,
        grid=(GRID,),
        in_specs=[
            pl.BlockSpec((BLK, D), lambda i: (i, 0)),
            pl.BlockSpec((D, H), lambda i: (0, 0)),
            pl.BlockSpec((BLK, 1), lambda i: (i, 0)),
        ],
        out_specs=pl.BlockSpec((NC, BLK, HALF), lambda i: (0, i, 0)),
        out_shape=jax.ShapeDtypeStruct((NC, N, HALF), jnp.float32),
    )(x, w1, dinv)


def _l2_body(acc_ref, hp_ref, dinv_ref, b1_ref, w2_ref, h2p_ref):
    a = jnp.concatenate([acc_ref[0], acc_ref[1]], axis=-1)
    hp = jnp.concatenate([hp_ref[0], hp_ref[1]], axis=-1)
    dinv = dinv_ref[...]
    h1 = jnp.maximum(dinv * (a + hp) + b1_ref[...], 0.0)
    h2p_ref[...] = dinv * jnp.dot(h1, w2_ref[...],
                                  preferred_element_type=jnp.float32)


def _tc_l2(acc1, hp, dinv, b1, w2p):
    return pl.pallas_call(
        _l2_body,
        grid=(GRID,),
        in_specs=[
            pl.BlockSpec((NC, BLK, HALF), lambda i: (0, i, 0)),
            pl.BlockSpec((NC, BLK, HALF), lambda i: (0, i, 0)),
            pl.BlockSpec((BLK, 1), lambda i: (i, 0)),
            pl.BlockSpec((1, H), lambda i: (0, 0)),
            pl.BlockSpec((H, C_PAD), lambda i: (0, 0)),
        ],
        out_specs=pl.BlockSpec((BLK, C_PAD), lambda i: (i, 0)),
        out_shape=jax.ShapeDtypeStruct((N, C_PAD), jnp.float32),
    )(acc1, hp, dinv, b1, w2p)


def _final_body(acc_ref, h2p_ref, dinv_ref, b2_ref, out_ref):
    a = acc_ref[0] + acc_ref[1] + h2p_ref[...]
    o = (dinv_ref[...] * a)[:, :C] + b2_ref[...]
    m = jnp.max(o, axis=1, keepdims=True)
    z = o - m
    lse = jnp.log(jnp.sum(jnp.exp(z), axis=1, keepdims=True))
    out_ref[...] = z - lse


def _tc_final(acc2, h2p, dinv, b2):
    return pl.pallas_call(
        _final_body,
        grid=(GRID,),
        in_specs=[
            pl.BlockSpec((NC, BLK, C_PAD), lambda i: (0, i, 0)),
            pl.BlockSpec((BLK, C_PAD), lambda i: (i, 0)),
            pl.BlockSpec((BLK, 1), lambda i: (i, 0)),
            pl.BlockSpec((1, C), lambda i: (0, 0)),
        ],
        out_specs=pl.BlockSpec((BLK, C), lambda i: (i, 0)),
        out_shape=jax.ShapeDtypeStruct((N, C), jnp.float32),
    )(acc2, h2p, dinv, b2)


@jax.jit
def kernel(x, edge_index, W1, b1, W2, b2):
    npad = E_PAD - E
    srcp = jnp.concatenate(
        [edge_index[0], jnp.zeros((npad,), jnp.int32)]).reshape(NCHUNK, CHUNK)
    dstp = jnp.concatenate(
        [edge_index[1], jnp.full((npad,), N, jnp.int32)]).reshape(NCHUNK, CHUNK)

    ones128 = jnp.ones((CHUNK, 128), jnp.float32)
    zero128 = jnp.zeros((RL, 128), jnp.float32)
    w2p = jnp.zeros((H, C_PAD), jnp.float32).at[:, :C].set(W2)

    degp = _sc_degree(dstp, ones128, zero128)
    dinv = _tc_dinv(degp)
    hp = _tc_l1(x, W1, dinv)
    idxp = jnp.stack([srcp, dstp], axis=1)
    acc1 = _sc_prop1(hp.reshape(NC * N, HALF), idxp, zero128)
    h2p = _tc_l2(acc1, hp, dinv, b1.reshape(1, H), w2p)
    acc2 = _sc_prop2(h2p, idxp, zero128)
    return _tc_final(acc2, h2p, dinv, b2.reshape(1, C))


# speedup vs baseline: 1.0719x; 1.0719x over previous
import functools

import jax
import jax.numpy as jnp
from jax import lax
from jax.experimental import pallas as pl
from jax.experimental.pallas import tpu as pltpu
from jax.experimental.pallas import tpu_sc as plsc

N = 10000
E = 160000
D = 256
H = 256
C = 40
C_PAD = 128
HALF = 128

NC = 2
NS = 16
CHUNK = 128
NCHUNK = 1280
E_PAD = NCHUNK * CHUNK
CPT1 = NCHUNK // NS
CPT2 = NCHUNK // (NC * NS)
NBUF = 4
R0 = 624
RL = N - (NS - 1) * R0

BLK = 400
GRID = N // BLK

_mesh = plsc.VectorSubcoreMesh(
    core_axis_name="c", subcore_axis_name="s", num_cores=NC, num_subcores=NS
)


def _zero_fill(zero_hbm, acc, s):
    @pl.when(s < NS - 1)
    def _():
        pltpu.sync_copy(zero_hbm.at[pl.ds(0, R0)], acc.at[pl.ds(s * R0, R0)])

    @pl.when(s == NS - 1)
    def _():
        pltpu.sync_copy(zero_hbm, acc.at[pl.ds((NS - 1) * R0, RL)])


def _writeback(acc, out_hbm, c, s):
    @pl.when(s < NS - 1)
    def _():
        pltpu.sync_copy(acc.at[pl.ds(s * R0, R0)],
                        out_hbm.at[c, pl.ds(s * R0, R0)])

    @pl.when(s == NS - 1)
    def _():
        pltpu.sync_copy(acc.at[pl.ds((NS - 1) * R0, RL)],
                        out_hbm.at[c, pl.ds((NS - 1) * R0, RL)])

@functools.partial(
    pl.kernel,
    out_type=jax.ShapeDtypeStruct((NC, N, 128), jnp.float32),
    mesh=_mesh,
    scratch_types=[
        pltpu.VMEM((CPT2, CHUNK), jnp.int32),
        pltpu.VMEM((CHUNK, 128), jnp.float32),
        pltpu.VMEM_SHARED((N + 128, 128), jnp.float32),
    ] + [pltpu.SemaphoreType.DMA] * NBUF,
)
def _sc_degree(dst_hbm, ones_hbm, zero_hbm, out_hbm, idx_v, ones_v, acc,
               s0, s1, s2, s3):
    c = lax.axis_index("c")
    s = lax.axis_index("s")
    ssem = (s0, s1, s2, s3)
    base = c * (NCHUNK // NC) + s * CPT2
    pltpu.sync_copy(dst_hbm.at[pl.ds(base, CPT2)], idx_v)
    pltpu.sync_copy(ones_hbm, ones_v)
    _zero_fill(zero_hbm, acc, s)
    plsc.subcore_barrier()

    for b in range(NBUF):
        pltpu.async_copy(ones_v, acc.at[idx_v.at[b]], ssem[b], add=True)

    def body(g, carry):
        for b in range(NBUF):
            _drain_scatter(ones_v, acc, idx_v.at[0], ssem[b])
            pltpu.async_copy(ones_v, acc.at[idx_v.at[g * NBUF + b]],
                             ssem[b], add=True)
        return carry

    lax.fori_loop(1, CPT2 // NBUF, body, 0)
    for b in range(NBUF):
        _drain_scatter(ones_v, acc, idx_v.at[0], ssem[b])
    plsc.subcore_barrier()
    _writeback(acc, out_hbm, c, s)


def _drain_gather(table_hbm, rows_v, b, sem):
    pltpu.make_async_copy(table_hbm.at[pl.ds(0, CHUNK)], rows_v.at[b],
                          sem).wait()


def _drain_scatter(ones_v, acc, idx_row, sem):
    pltpu.make_async_copy(ones_v, acc.at[idx_row], sem).wait()


def _make_prop(split_edges):
    cpt = CPT2 if split_edges else CPT1

    @functools.partial(
        pl.kernel,
        out_type=jax.ShapeDtypeStruct((NC, N, 128), jnp.float32),
        mesh=_mesh,
        scratch_types=[
            pltpu.VMEM((4, 2, CHUNK), jnp.int32),
            pltpu.VMEM((2, CHUNK, 128), jnp.float32),
            pltpu.VMEM_SHARED((N + 128, 128), jnp.float32),
        ] + [pltpu.SemaphoreType.DMA] * 8,
    )
    def prop(tbl_hbm, idx_hbm, zero_hbm, out_hbm,
             idx_v, rows_v, acc, g0, g1, t0, t1, i0, i1, i2, i3):
        c = lax.axis_index("c")
        s = lax.axis_index("s")
        gsem = (g0, g1)
        ssem = (t0, t1)
        isem = (i0, i1, i2, i3)
        if split_edges:
            base = c * (NCHUNK // NC) + s * cpt
        else:
            base = s * cpt

        def issue_idx(j, q):
            pltpu.async_copy(idx_hbm.at[base + j], idx_v.at[q], isem[q])

        def wait_idx(q):
            pltpu.make_async_copy(idx_hbm.at[0], idx_v.at[q], isem[q]).wait()

        def issue_gather(q, b):
            if not split_edges:
                @pl.when(c > 0)
                def _():
                    for tt in range(CHUNK // 16):
                        sl = pl.ds(tt * 16, 16)
                        idx_v[q, 0, sl] = idx_v[q, 0, sl] + N
            pltpu.async_copy(tbl_hbm.at[idx_v.at[q, 0]], rows_v.at[b],
                             gsem[b])

        def wait_gather(b):
            pltpu.make_async_copy(tbl_hbm.at[idx_v.at[0, 0]], rows_v.at[b],
                                  gsem[b]).wait()

        def issue_scatter(q, b):
            pltpu.async_copy(rows_v.at[b], acc.at[idx_v.at[q, 1]], ssem[b],
                             add=True)

        def wait_scatter(b):
            pltpu.make_async_copy(rows_v.at[b], acc.at[idx_v.at[0, 1]],
                                  ssem[b]).wait()

        for q in range(4):
            issue_idx(q, q)
        _zero_fill(zero_hbm, acc, s)
        plsc.subcore_barrier()
        for b in range(2):
            wait_idx(b)
            issue_gather(b, b)

        def body(g, carry):
            j0 = g * 4
            for u in range(4):
                j = j0 + u
                b = u % 2
                wait_gather(b)
                issue_scatter(u, b)
                wait_scatter(b)
                issue_idx(j + 4, u)
                wait_idx((u + 2) % 4)
                issue_gather((u + 2) % 4, b)
            return carry

        lax.fori_loop(0, cpt // 4 - 1, body, 0)
        for u in range(4):
            b = u % 2
            wait_gather(b)
            issue_scatter(u, b)
            wait_scatter(b)
            if u < 2:
                wait_idx((u + 2) % 4)
                issue_gather((u + 2) % 4, b)
        plsc.subcore_barrier()
        _writeback(acc, out_hbm, c, s)

    return prop


_sc_prop1 = _make_prop(split_edges=False)
_sc_prop2 = _make_prop(split_edges=True)


def _dinv_body(degp_ref, dinv_ref):
    deg = degp_ref[0, :, 0] + degp_ref[1, :, 0] + 1.0
    dinv_ref[:, 0] = 1.0 / jnp.sqrt(deg)


def _tc_dinv(degp):
    return pl.pallas_call(
        _dinv_body,
        out_shape=jax.ShapeDtypeStruct((N, 1), jnp.float32),
    )(degp)


def _l1_body(x_ref, w1_ref, dinv_ref, hp_ref):
    h = jnp.dot(x_ref[...], w1_ref[...], preferred_element_type=jnp.float32)
    hp = dinv_ref[...] * h
    hp_ref[0] = hp[:, :HALF]
    hp_ref[1] = hp[:, HALF:]


def _tc_l1(x, w1, dinv):
    return pl.pallas_call(
        _l1_body,
        grid=(GRID,),
        in_specs=[
            pl.BlockSpec((BLK, D), lambda i: (i, 0)),
            pl.BlockSpec((D, H), lambda i: (0, 0)),
            pl.BlockSpec((BLK, 1), lambda i: (i, 0)),
        ],
        out_specs=pl.BlockSpec((NC, BLK, HALF), lambda i: (0, i, 0)),
        out_shape=jax.ShapeDtypeStruct((NC, N, HALF), jnp.float32),
    )(x, w1, dinv)


def _l2_body(acc_ref, hp_ref, dinv_ref, b1_ref, w2_ref, h2p_ref):
    a = jnp.concatenate([acc_ref[0], acc_ref[1]], axis=-1)
    hp = jnp.concatenate([hp_ref[0], hp_ref[1]], axis=-1)
    dinv = dinv_ref[...]
    h1 = jnp.maximum(dinv * (a + hp) + b1_ref[...], 0.0)
    h2p_ref[...] = dinv * jnp.dot(h1, w2_ref[...],
                                  preferred_element_type=jnp.float32)


def _tc_l2(acc1, hp, dinv, b1, w2p):
    return pl.pallas_call(
        _l2_body,
        grid=(GRID,),
        in_specs=[
            pl.BlockSpec((NC, BLK, HALF), lambda i: (0, i, 0)),
            pl.BlockSpec((NC, BLK, HALF), lambda i: (0, i, 0)),
            pl.BlockSpec((BLK, 1), lambda i: (i, 0)),
            pl.BlockSpec((1, H), lambda i: (0, 0)),
            pl.BlockSpec((H, C_PAD), lambda i: (0, 0)),
        ],
        out_specs=pl.BlockSpec((BLK, C_PAD), lambda i: (i, 0)),
        out_shape=jax.ShapeDtypeStruct((N, C_PAD), jnp.float32),
    )(acc1, hp, dinv, b1, w2p)


def _final_body(acc_ref, h2p_ref, dinv_ref, b2_ref, out_ref):
    a = acc_ref[0] + acc_ref[1] + h2p_ref[...]
    o = (dinv_ref[...] * a)[:, :C] + b2_ref[...]
    m = jnp.max(o, axis=1, keepdims=True)
    z = o - m
    lse = jnp.log(jnp.sum(jnp.exp(z), axis=1, keepdims=True))
    out_ref[...] = z - lse


def _tc_final(acc2, h2p, dinv, b2):
    return pl.pallas_call(
        _final_body,
        grid=(GRID,),
        in_specs=[
            pl.BlockSpec((NC, BLK, C_PAD), lambda i: (0, i, 0)),
            pl.BlockSpec((BLK, C_PAD), lambda i: (i, 0)),
            pl.BlockSpec((BLK, 1), lambda i: (i, 0)),
            pl.BlockSpec((1, C), lambda i: (0, 0)),
        ],
        out_specs=pl.BlockSpec((BLK, C), lambda i: (i, 0)),
        out_shape=jax.ShapeDtypeStruct((N, C), jnp.float32),
    )(acc2, h2p, dinv, b2)


@jax.jit
def kernel(x, edge_index, W1, b1, W2, b2):
    npad = E_PAD - E
    srcp = jnp.concatenate(
        [edge_index[0], jnp.zeros((npad,), jnp.int32)]).reshape(NCHUNK, CHUNK)
    dstp = jnp.concatenate(
        [edge_index[1],
         N + (jnp.arange(npad, dtype=jnp.int32) % 128)]).reshape(NCHUNK, CHUNK)

    ones128 = jnp.ones((CHUNK, 128), jnp.float32)
    zero128 = jnp.zeros((RL, 128), jnp.float32)
    w2p = jnp.zeros((H, C_PAD), jnp.float32).at[:, :C].set(W2)

    degp = _sc_degree(dstp, ones128, zero128)
    dinv = _tc_dinv(degp)
    hp = _tc_l1(x, W1, dinv)
    idxp = jnp.stack([srcp, dstp], axis=1)
    acc1 = _sc_prop1(hp.reshape(NC * N, HALF), idxp, zero128)
    h2p = _tc_l2(acc1, hp, dinv, b1.reshape(1, H), w2p)
    acc2 = _sc_prop2(h2p, idxp, zero128)
    return _tc_final(acc2, h2p, dinv, b2.reshape(1, C))


# speedup vs baseline: 2.2599x; 2.1082x over previous
import functools

import jax
import jax.numpy as jnp
from jax import lax
from jax.experimental import pallas as pl
from jax.experimental.pallas import tpu as pltpu
from jax.experimental.pallas import tpu_sc as plsc

N = 10000
E = 160000
D = 256
H = 256
C = 40
C_PAD = 128
HALF = 128

NC = 2
NS = 16
CHUNK = 128
NCHUNK = 1280
E_PAD = NCHUNK * CHUNK
CPT1 = NCHUNK // NS
CPT2 = NCHUNK // (NC * NS)
NBUF = 4
R0 = 624
RL = N - (NS - 1) * R0

BLK = 400
GRID = N // BLK

_mesh = plsc.VectorSubcoreMesh(
    core_axis_name="c", subcore_axis_name="s", num_cores=NC, num_subcores=NS
)


def _zero_fill(zero_hbm, acc, s):
    @pl.when(s < NS - 1)
    def _():
        pltpu.sync_copy(zero_hbm.at[pl.ds(0, R0)], acc.at[pl.ds(s * R0, R0)])

    @pl.when(s == NS - 1)
    def _():
        pltpu.sync_copy(zero_hbm, acc.at[pl.ds((NS - 1) * R0, RL)])


def _writeback(acc, out_hbm, c, s):
    @pl.when(s < NS - 1)
    def _():
        pltpu.sync_copy(acc.at[pl.ds(s * R0, R0)],
                        out_hbm.at[c, pl.ds(s * R0, R0)])

    @pl.when(s == NS - 1)
    def _():
        pltpu.sync_copy(acc.at[pl.ds((NS - 1) * R0, RL)],
                        out_hbm.at[c, pl.ds((NS - 1) * R0, RL)])

@functools.partial(
    pl.kernel,
    out_type=jax.ShapeDtypeStruct((NC, N, 128), jnp.float32),
    mesh=_mesh,
    scratch_types=[
        pltpu.VMEM((CPT2, CHUNK), jnp.int32),
        pltpu.VMEM((CHUNK, 128), jnp.float32),
        pltpu.VMEM_SHARED((N + 128, 128), jnp.float32),
    ] + [pltpu.SemaphoreType.DMA] * NBUF,
)
def _sc_degree(dst_hbm, ones_hbm, zero_hbm, out_hbm, idx_v, ones_v, acc,
               s0, s1, s2, s3):
    c = lax.axis_index("c")
    s = lax.axis_index("s")
    ssem = (s0, s1, s2, s3)
    base = c * (NCHUNK // NC) + s * CPT2
    pltpu.sync_copy(dst_hbm.at[pl.ds(base, CPT2)], idx_v)
    pltpu.sync_copy(ones_hbm, ones_v)
    _zero_fill(zero_hbm, acc, s)
    plsc.subcore_barrier()

    for b in range(NBUF):
        pltpu.async_copy(ones_v, acc.at[idx_v.at[b]], ssem[b], add=True)

    def body(g, carry):
        for b in range(NBUF):
            _drain_scatter(ones_v, acc, idx_v.at[0], ssem[b])
            pltpu.async_copy(ones_v, acc.at[idx_v.at[g * NBUF + b]],
                             ssem[b], add=True)
        return carry

    lax.fori_loop(1, CPT2 // NBUF, body, 0)
    for b in range(NBUF):
        _drain_scatter(ones_v, acc, idx_v.at[0], ssem[b])
    plsc.subcore_barrier()
    _writeback(acc, out_hbm, c, s)


def _drain_gather(table_hbm, rows_v, b, sem):
    pltpu.make_async_copy(table_hbm.at[pl.ds(0, CHUNK)], rows_v.at[b],
                          sem).wait()


def _drain_scatter(ones_v, acc, idx_row, sem):
    pltpu.make_async_copy(ones_v, acc.at[idx_row], sem).wait()


def _make_prop(split_edges):
    cpt = CPT2 if split_edges else CPT1

    @functools.partial(
        pl.kernel,
        out_type=jax.ShapeDtypeStruct((NC, N, 128), jnp.float32),
        mesh=_mesh,
        scratch_types=[
            pltpu.VMEM((4, 2, CHUNK), jnp.int32),
            pltpu.VMEM((2, CHUNK, 128), jnp.float32),
            pltpu.VMEM_SHARED((N + 128, 128), jnp.float32),
        ] + [pltpu.SemaphoreType.DMA] * 8,
    )
    def prop(tbl_hbm, idx_hbm, zero_hbm, out_hbm,
             idx_v, rows_v, acc, g0, g1, t0, t1, i0, i1, i2, i3):
        c = lax.axis_index("c")
        s = lax.axis_index("s")
        gsem = (g0, g1)
        ssem = (t0, t1)
        isem = (i0, i1, i2, i3)
        if split_edges:
            base = c * (NCHUNK // NC) + s * cpt
        else:
            base = s * cpt

        def issue_idx(j, q):
            pltpu.async_copy(idx_hbm.at[base + j], idx_v.at[q], isem[q])

        def wait_idx(q):
            pltpu.make_async_copy(idx_hbm.at[0], idx_v.at[q], isem[q]).wait()

        def issue_gather(q, b):
            if not split_edges:
                @pl.when(c > 0)
                def _():
                    for tt in range(CHUNK // 16):
                        sl = pl.ds(tt * 16, 16)
                        idx_v[q, 0, sl] = idx_v[q, 0, sl] + N
            pltpu.async_copy(tbl_hbm.at[idx_v.at[q, 0]], rows_v.at[b],
                             gsem[b])

        def wait_gather(b):
            pltpu.make_async_copy(tbl_hbm.at[idx_v.at[0, 0]], rows_v.at[b],
                                  gsem[b]).wait()

        def issue_scatter(q, b):
            pltpu.async_copy(rows_v.at[b], acc.at[idx_v.at[q, 1]], ssem[b],
                             add=True)

        def wait_scatter(b):
            pltpu.make_async_copy(rows_v.at[b], acc.at[idx_v.at[0, 1]],
                                  ssem[b]).wait()

        for q in range(4):
            issue_idx(q, q)
        _zero_fill(zero_hbm, acc, s)
        plsc.subcore_barrier()
        for b in range(2):
            wait_idx(b)
            issue_gather(b, b)

        def body(g, carry):
            j0 = g * 4
            for u in range(4):
                j = j0 + u
                b = u % 2
                wait_gather(b)
                issue_scatter(u, b)
                wait_scatter(b)
                issue_idx(j + 4, u)
                wait_idx((u + 2) % 4)
                issue_gather((u + 2) % 4, b)
            return carry

        lax.fori_loop(0, cpt // 4 - 1, body, 0)
        for u in range(4):
            b = u % 2
            wait_gather(b)
            issue_scatter(u, b)
            wait_scatter(b)
            if u < 2:
                wait_idx((u + 2) % 4)
                issue_gather((u + 2) % 4, b)
        plsc.subcore_barrier()
        _writeback(acc, out_hbm, c, s)

    return prop


_sc_prop1 = _make_prop(split_edges=False)
_sc_prop2 = _make_prop(split_edges=True)


def _dinv_body(degp_ref, dinv_ref):
    deg = degp_ref[0, :, 0] + degp_ref[1, :, 0] + 1.0
    dinv_ref[:, 0] = 1.0 / jnp.sqrt(deg)


def _tc_dinv(degp):
    return pl.pallas_call(
        _dinv_body,
        out_shape=jax.ShapeDtypeStruct((N, 1), jnp.float32),
    )(degp)


def _l1_body(x_ref, w1_ref, dinv_ref, hp_ref):
    h = jnp.dot(x_ref[...], w1_ref[...], preferred_element_type=jnp.float32)
    hp = dinv_ref[...] * h
    hp_ref[0] = hp[:, :HALF]
    hp_ref[1] = hp[:, HALF:]


def _tc_l1(x, w1, dinv):
    return pl.pallas_call(
        _l1_body,
        grid=(GRID,),
        in_specs=[
            pl.BlockSpec((BLK, D), lambda i: (i, 0)),
            pl.BlockSpec((D, H), lambda i: (0, 0)),
            pl.BlockSpec((BLK, 1), lambda i: (i, 0)),
        ],
        out_specs=pl.BlockSpec((NC, BLK, HALF), lambda i: (0, i, 0)),
        out_shape=jax.ShapeDtypeStruct((NC, N, HALF), jnp.float32),
    )(x, w1, dinv)


def _l2_body(acc_ref, hp_ref, dinv_ref, b1_ref, w2_ref, h2p_ref):
    a = jnp.concatenate([acc_ref[0], acc_ref[1]], axis=-1)
    hp = jnp.concatenate([hp_ref[0], hp_ref[1]], axis=-1)
    dinv = dinv_ref[...]
    h1 = jnp.maximum(dinv * (a + hp) + b1_ref[...], 0.0)
    h2p_ref[...] = dinv * jnp.dot(h1, w2_ref[...],
                                  preferred_element_type=jnp.float32)


def _tc_l2(acc1, hp, dinv, b1, w2p):
    return pl.pallas_call(
        _l2_body,
        grid=(GRID,),
        in_specs=[
            pl.BlockSpec((NC, BLK, HALF), lambda i: (0, i, 0)),
            pl.BlockSpec((NC, BLK, HALF), lambda i: (0, i, 0)),
            pl.BlockSpec((BLK, 1), lambda i: (i, 0)),
            pl.BlockSpec((1, H), lambda i: (0, 0)),
            pl.BlockSpec((H, C_PAD), lambda i: (0, 0)),
        ],
        out_specs=pl.BlockSpec((BLK, C_PAD), lambda i: (i, 0)),
        out_shape=jax.ShapeDtypeStruct((N, C_PAD), jnp.float32),
    )(acc1, hp, dinv, b1, w2p)


def _final_body(acc_ref, h2p_ref, dinv_ref, b2_ref, out_ref):
    a = acc_ref[0] + acc_ref[1] + h2p_ref[...]
    o = (dinv_ref[...] * a)[:, :C] + b2_ref[...]
    m = jnp.max(o, axis=1, keepdims=True)
    z = o - m
    lse = jnp.log(jnp.sum(jnp.exp(z), axis=1, keepdims=True))
    out_ref[...] = z - lse


def _tc_final(acc2, h2p, dinv, b2):
    return pl.pallas_call(
        _final_body,
        grid=(GRID,),
        in_specs=[
            pl.BlockSpec((NC, BLK, C_PAD), lambda i: (0, i, 0)),
            pl.BlockSpec((BLK, C_PAD), lambda i: (i, 0)),
            pl.BlockSpec((BLK, 1), lambda i: (i, 0)),
            pl.BlockSpec((1, C), lambda i: (0, 0)),
        ],
        out_specs=pl.BlockSpec((BLK, C), lambda i: (i, 0)),
        out_shape=jax.ShapeDtypeStruct((N, C), jnp.float32),
    )(acc2, h2p, dinv, b2)


@jax.jit
def kernel(x, edge_index, W1, b1, W2, b2):
    npad = E_PAD - E
    srcp = jnp.concatenate(
        [edge_index[0],
         jnp.arange(npad, dtype=jnp.int32) % 128]).reshape(NCHUNK, CHUNK)
    dstp = jnp.concatenate(
        [edge_index[1],
         N + (jnp.arange(npad, dtype=jnp.int32) % 128)]).reshape(NCHUNK, CHUNK)

    ones128 = jnp.ones((CHUNK, 128), jnp.float32)
    zero128 = jnp.zeros((RL, 128), jnp.float32)
    w2p = jnp.zeros((H, C_PAD), jnp.float32).at[:, :C].set(W2)

    degp = _sc_degree(dstp, ones128, zero128)
    dinv = _tc_dinv(degp)
    hp = _tc_l1(x, W1, dinv)
    idxp = jnp.stack([srcp, dstp], axis=1)
    acc1 = _sc_prop1(hp.reshape(NC * N, HALF), idxp, zero128)
    h2p = _tc_l2(acc1, hp, dinv, b1.reshape(1, H), w2p)
    acc2 = _sc_prop2(h2p, idxp, zero128)
    return _tc_final(acc2, h2p, dinv, b2.reshape(1, C))


# speedup vs baseline: 2.2692x; 1.0041x over previous
import functools

import jax
import jax.numpy as jnp
from jax import lax
from jax.experimental import pallas as pl
from jax.experimental.pallas import tpu as pltpu
from jax.experimental.pallas import tpu_sc as plsc

N = 10000
E = 160000
D = 256
H = 256
C = 40
C_PAD = 128
HALF = 128

NC = 2
NS = 16
CHUNK = 128
NCHUNK = 1280
E_PAD = NCHUNK * CHUNK
CPT1 = NCHUNK // NS
CPT2 = NCHUNK // (NC * NS)
NBUF = 4
R0 = 624
RL = N - (NS - 1) * R0

BLK = 400
GRID = N // BLK

_mesh = plsc.VectorSubcoreMesh(
    core_axis_name="c", subcore_axis_name="s", num_cores=NC, num_subcores=NS
)


def _zero_fill(zero_hbm, acc, s):
    @pl.when(s < NS - 1)
    def _():
        pltpu.sync_copy(zero_hbm.at[pl.ds(0, R0)], acc.at[pl.ds(s * R0, R0)])

    @pl.when(s == NS - 1)
    def _():
        pltpu.sync_copy(zero_hbm, acc.at[pl.ds((NS - 1) * R0, RL)])


def _writeback(acc, out_hbm, c, s):
    @pl.when(s < NS - 1)
    def _():
        pltpu.sync_copy(acc.at[pl.ds(s * R0, R0)],
                        out_hbm.at[c, pl.ds(s * R0, R0)])

    @pl.when(s == NS - 1)
    def _():
        pltpu.sync_copy(acc.at[pl.ds((NS - 1) * R0, RL)],
                        out_hbm.at[c, pl.ds((NS - 1) * R0, RL)])

@functools.partial(
    pl.kernel,
    out_type=jax.ShapeDtypeStruct((NC, N, 128), jnp.float32),
    mesh=_mesh,
    scratch_types=[
        pltpu.VMEM((CPT2, CHUNK), jnp.int32),
        pltpu.VMEM((CHUNK, 128), jnp.float32),
        pltpu.VMEM_SHARED((N + 128, 128), jnp.float32),
    ] + [pltpu.SemaphoreType.DMA] * NBUF,
)
def _sc_degree(dst_hbm, ones_hbm, zero_hbm, out_hbm, idx_v, ones_v, acc,
               s0, s1, s2, s3):
    c = lax.axis_index("c")
    s = lax.axis_index("s")
    ssem = (s0, s1, s2, s3)
    base = c * (NCHUNK // NC) + s * CPT2
    pltpu.sync_copy(dst_hbm.at[pl.ds(base, CPT2)], idx_v)
    pltpu.sync_copy(ones_hbm, ones_v)
    _zero_fill(zero_hbm, acc, s)
    plsc.subcore_barrier()

    for b in range(NBUF):
        pltpu.async_copy(ones_v, acc.at[idx_v.at[b]], ssem[b], add=True)

    def body(g, carry):
        for b in range(NBUF):
            _drain_scatter(ones_v, acc, idx_v.at[0], ssem[b])
            pltpu.async_copy(ones_v, acc.at[idx_v.at[g * NBUF + b]],
                             ssem[b], add=True)
        return carry

    lax.fori_loop(1, CPT2 // NBUF, body, 0)
    for b in range(NBUF):
        _drain_scatter(ones_v, acc, idx_v.at[0], ssem[b])
    plsc.subcore_barrier()
    _writeback(acc, out_hbm, c, s)


def _drain_gather(table_hbm, rows_v, b, sem):
    pltpu.make_async_copy(table_hbm.at[pl.ds(0, CHUNK)], rows_v.at[b],
                          sem).wait()


def _drain_scatter(ones_v, acc, idx_row, sem):
    pltpu.make_async_copy(ones_v, acc.at[idx_row], sem).wait()


def _make_prop(split_edges):
    cpt = CPT2 if split_edges else CPT1

    @functools.partial(
        pl.kernel,
        out_type=jax.ShapeDtypeStruct((NC, N, 128), jnp.float32),
        mesh=_mesh,
        scratch_types=[
            pltpu.VMEM((4, 2, CHUNK), jnp.int32),
            pltpu.VMEM((2, CHUNK, 128), jnp.float32),
            pltpu.VMEM_SHARED((N + 128, 128), jnp.float32),
        ] + [pltpu.SemaphoreType.DMA] * 8,
    )
    def prop(tbl_hbm, idx_hbm, zero_hbm, out_hbm,
             idx_v, rows_v, acc, g0, g1, t0, t1, i0, i1, i2, i3):
        c = lax.axis_index("c")
        s = lax.axis_index("s")
        gsem = (g0, g1)
        ssem = (t0, t1)
        isem = (i0, i1, i2, i3)
        if split_edges:
            base = c * (NCHUNK // NC) + s * cpt
        else:
            base = s * cpt

        def issue_idx(j, q):
            pltpu.async_copy(idx_hbm.at[base + j], idx_v.at[q], isem[q])

        def wait_idx(q):
            pltpu.make_async_copy(idx_hbm.at[0], idx_v.at[q], isem[q]).wait()

        def issue_gather(q, b):
            if not split_edges:
                @pl.when(c > 0)
                def _():
                    for tt in range(CHUNK // 16):
                        sl = pl.ds(tt * 16, 16)
                        idx_v[q, 0, sl] = idx_v[q, 0, sl] + N
            pltpu.async_copy(tbl_hbm.at[idx_v.at[q, 0]], rows_v.at[b],
                             gsem[b])

        def wait_gather(b):
            pltpu.make_async_copy(tbl_hbm.at[idx_v.at[0, 0]], rows_v.at[b],
                                  gsem[b]).wait()

        def issue_scatter(q, b):
            pltpu.async_copy(rows_v.at[b], acc.at[idx_v.at[q, 1]], ssem[b],
                             add=True)

        def wait_scatter(b):
            pltpu.make_async_copy(rows_v.at[b], acc.at[idx_v.at[0, 1]],
                                  ssem[b]).wait()

        for q in range(4):
            issue_idx(q, q)
        _zero_fill(zero_hbm, acc, s)
        plsc.subcore_barrier()
        for b in range(2):
            wait_idx(b)
            issue_gather(b, b)

        def body(g, carry):
            j0 = g * 4
            for u in range(4):
                j = j0 + u
                b = u % 2
                wait_gather(b)
                issue_scatter(u, b)
                wait_scatter(b)
                issue_idx(j + 4, u)
                wait_idx((u + 2) % 4)
                issue_gather((u + 2) % 4, b)
            return carry

        lax.fori_loop(0, cpt // 4 - 1, body, 0)
        for u in range(4):
            b = u % 2
            wait_gather(b)
            issue_scatter(u, b)
            wait_scatter(b)
            if u < 2:
                wait_idx((u + 2) % 4)
                issue_gather((u + 2) % 4, b)
        plsc.subcore_barrier()
        _writeback(acc, out_hbm, c, s)

    return prop


_sc_prop1 = _make_prop(split_edges=False)
_sc_prop2 = _make_prop(split_edges=True)


def _l1_body(x_ref, w1_ref, degp_ref, hp_ref, dinv_ref):
    deg = degp_ref[0, :, 0] + degp_ref[1, :, 0] + 1.0
    dinv = (1.0 / jnp.sqrt(deg))[:, None]
    dinv_ref[...] = dinv
    h = jnp.dot(x_ref[...], w1_ref[...], preferred_element_type=jnp.float32)
    hp = dinv * h
    hp_ref[0] = hp[:, :HALF]
    hp_ref[1] = hp[:, HALF:]


def _tc_l1(x, w1, degp):
    return pl.pallas_call(
        _l1_body,
        grid=(GRID,),
        in_specs=[
            pl.BlockSpec((BLK, D), lambda i: (i, 0)),
            pl.BlockSpec((D, H), lambda i: (0, 0)),
            pl.BlockSpec((NC, BLK, 128), lambda i: (0, i, 0)),
        ],
        out_specs=[
            pl.BlockSpec((NC, BLK, HALF), lambda i: (0, i, 0)),
            pl.BlockSpec((BLK, 1), lambda i: (i, 0)),
        ],
        out_shape=[
            jax.ShapeDtypeStruct((NC, N, HALF), jnp.float32),
            jax.ShapeDtypeStruct((N, 1), jnp.float32),
        ],
    )(x, w1, degp)


def _l2_body(acc_ref, hp_ref, dinv_ref, b1_ref, w2_ref, h2p_ref):
    a = jnp.concatenate([acc_ref[0], acc_ref[1]], axis=-1)
    hp = jnp.concatenate([hp_ref[0], hp_ref[1]], axis=-1)
    dinv = dinv_ref[...]
    h1 = jnp.maximum(dinv * (a + hp) + b1_ref[...], 0.0)
    h2p_ref[...] = dinv * jnp.dot(h1, w2_ref[...],
                                  preferred_element_type=jnp.float32)


def _tc_l2(acc1, hp, dinv, b1, w2p):
    return pl.pallas_call(
        _l2_body,
        grid=(GRID,),
        in_specs=[
            pl.BlockSpec((NC, BLK, HALF), lambda i: (0, i, 0)),
            pl.BlockSpec((NC, BLK, HALF), lambda i: (0, i, 0)),
            pl.BlockSpec((BLK, 1), lambda i: (i, 0)),
            pl.BlockSpec((1, H), lambda i: (0, 0)),
            pl.BlockSpec((H, C_PAD), lambda i: (0, 0)),
        ],
        out_specs=pl.BlockSpec((BLK, C_PAD), lambda i: (i, 0)),
        out_shape=jax.ShapeDtypeStruct((N, C_PAD), jnp.float32),
    )(acc1, hp, dinv, b1, w2p)


def _final_body(acc_ref, h2p_ref, dinv_ref, b2_ref, out_ref):
    a = acc_ref[0] + acc_ref[1] + h2p_ref[...]
    o = (dinv_ref[...] * a)[:, :C] + b2_ref[...]
    m = jnp.max(o, axis=1, keepdims=True)
    z = o - m
    lse = jnp.log(jnp.sum(jnp.exp(z), axis=1, keepdims=True))
    out_ref[...] = z - lse


def _tc_final(acc2, h2p, dinv, b2):
    return pl.pallas_call(
        _final_body,
        grid=(GRID,),
        in_specs=[
            pl.BlockSpec((NC, BLK, C_PAD), lambda i: (0, i, 0)),
            pl.BlockSpec((BLK, C_PAD), lambda i: (i, 0)),
            pl.BlockSpec((BLK, 1), lambda i: (i, 0)),
            pl.BlockSpec((1, C), lambda i: (0, 0)),
        ],
        out_specs=pl.BlockSpec((BLK, C), lambda i: (i, 0)),
        out_shape=jax.ShapeDtypeStruct((N, C), jnp.float32),
    )(acc2, h2p, dinv, b2)


@jax.jit
def kernel(x, edge_index, W1, b1, W2, b2):
    npad = E_PAD - E
    srcp = jnp.concatenate(
        [edge_index[0],
         jnp.arange(npad, dtype=jnp.int32) % 128]).reshape(NCHUNK, CHUNK)
    dstp = jnp.concatenate(
        [edge_index[1],
         N + (jnp.arange(npad, dtype=jnp.int32) % 128)]).reshape(NCHUNK, CHUNK)

    ones128 = jnp.ones((CHUNK, 128), jnp.float32)
    zero128 = jnp.zeros((RL, 128), jnp.float32)
    w2p = jnp.zeros((H, C_PAD), jnp.float32).at[:, :C].set(W2)

    degp = _sc_degree(dstp, ones128, zero128)
    hp, dinv = _tc_l1(x, W1, degp)
    idxp = jnp.stack([srcp, dstp], axis=1)
    acc1 = _sc_prop1(hp.reshape(NC * N, HALF), idxp, zero128)
    h2p = _tc_l2(acc1, hp, dinv, b1.reshape(1, H), w2p)
    acc2 = _sc_prop2(h2p, idxp, zero128)
    return _tc_final(acc2, h2p, dinv, b2.reshape(1, C))


# speedup vs baseline: 2.4527x; 1.0809x over previous
import functools

import jax
import jax.numpy as jnp
from jax import lax
from jax.experimental import pallas as pl
from jax.experimental.pallas import tpu as pltpu
from jax.experimental.pallas import tpu_sc as plsc

N = 10000
E = 160000
D = 256
H = 256
C = 40
C_PAD = 128
HALF = 128

NC = 2
NS = 16
CHUNK = 64
NCHUNK = 2560
E_PAD = NCHUNK * CHUNK
CPT1 = NCHUNK // NS
CPT2 = NCHUNK // (NC * NS)
NBUF = 4
R0 = 624
RL = N - (NS - 1) * R0

BLK = 400
GRID = N // BLK

_mesh = plsc.VectorSubcoreMesh(
    core_axis_name="c", subcore_axis_name="s", num_cores=NC, num_subcores=NS
)


def _zero_fill(zero_hbm, acc, s):
    @pl.when(s < NS - 1)
    def _():
        pltpu.sync_copy(zero_hbm.at[pl.ds(0, R0)], acc.at[pl.ds(s * R0, R0)])

    @pl.when(s == NS - 1)
    def _():
        pltpu.sync_copy(zero_hbm, acc.at[pl.ds((NS - 1) * R0, RL)])


def _writeback(acc, out_hbm, c, s):
    @pl.when(s < NS - 1)
    def _():
        pltpu.sync_copy(acc.at[pl.ds(s * R0, R0)],
                        out_hbm.at[c, pl.ds(s * R0, R0)])

    @pl.when(s == NS - 1)
    def _():
        pltpu.sync_copy(acc.at[pl.ds((NS - 1) * R0, RL)],
                        out_hbm.at[c, pl.ds((NS - 1) * R0, RL)])

@functools.partial(
    pl.kernel,
    out_type=jax.ShapeDtypeStruct((NC, N, 128), jnp.float32),
    mesh=_mesh,
    scratch_types=[
        pltpu.VMEM((CPT2, CHUNK), jnp.int32),
        pltpu.VMEM((CHUNK, 128), jnp.float32),
        pltpu.VMEM_SHARED((N + 128, 128), jnp.float32),
    ] + [pltpu.SemaphoreType.DMA] * NBUF,
)
def _sc_degree(dst_hbm, ones_hbm, zero_hbm, out_hbm, idx_v, ones_v, acc,
               s0, s1, s2, s3):
    c = lax.axis_index("c")
    s = lax.axis_index("s")
    ssem = (s0, s1, s2, s3)
    base = c * (NCHUNK // NC) + s * CPT2
    pltpu.sync_copy(dst_hbm.at[pl.ds(base, CPT2)], idx_v)
    pltpu.sync_copy(ones_hbm, ones_v)
    _zero_fill(zero_hbm, acc, s)
    plsc.subcore_barrier()

    for b in range(NBUF):
        pltpu.async_copy(ones_v, acc.at[idx_v.at[b]], ssem[b], add=True)

    def body(g, carry):
        for b in range(NBUF):
            _drain_scatter(ones_v, acc, idx_v.at[0], ssem[b])
            pltpu.async_copy(ones_v, acc.at[idx_v.at[g * NBUF + b]],
                             ssem[b], add=True)
        return carry

    lax.fori_loop(1, CPT2 // NBUF, body, 0)
    for b in range(NBUF):
        _drain_scatter(ones_v, acc, idx_v.at[0], ssem[b])
    plsc.subcore_barrier()
    _writeback(acc, out_hbm, c, s)


def _drain_gather(table_hbm, rows_v, b, sem):
    pltpu.make_async_copy(table_hbm.at[pl.ds(0, CHUNK)], rows_v.at[b],
                          sem).wait()


def _drain_scatter(ones_v, acc, idx_row, sem):
    pltpu.make_async_copy(ones_v, acc.at[idx_row], sem).wait()


def _make_prop(split_edges):
    cpt = CPT2 if split_edges else CPT1

    @functools.partial(
        pl.kernel,
        out_type=jax.ShapeDtypeStruct((NC, N, 128), jnp.float32),
        mesh=_mesh,
        scratch_types=[
            pltpu.VMEM((8, 2, CHUNK), jnp.int32),
            pltpu.VMEM((4, CHUNK, 128), jnp.float32),
            pltpu.VMEM_SHARED((N + 128, 128), jnp.float32),
        ] + [pltpu.SemaphoreType.DMA] * 16,
    )
    def prop(tbl_hbm, idx_hbm, zero_hbm, out_hbm,
             idx_v, rows_v, acc, g0, g1, g2, g3, t0, t1, t2, t3,
             i0, i1, i2, i3, i4, i5, i6, i7):
        c = lax.axis_index("c")
        s = lax.axis_index("s")
        gsem = (g0, g1, g2, g3)
        ssem = (t0, t1, t2, t3)
        isem = (i0, i1, i2, i3, i4, i5, i6, i7)
        if split_edges:
            base = c * (NCHUNK // NC) + s * cpt
        else:
            base = s * cpt

        def issue_idx(j, q):
            pltpu.async_copy(idx_hbm.at[base + j], idx_v.at[q], isem[q])

        def wait_idx(q):
            pltpu.make_async_copy(idx_hbm.at[0], idx_v.at[q], isem[q]).wait()

        def issue_gather(q, b):
            if not split_edges:
                @pl.when(c > 0)
                def _():
                    for tt in range(CHUNK // 16):
                        sl = pl.ds(tt * 16, 16)
                        idx_v[q, 0, sl] = idx_v[q, 0, sl] + N
            pltpu.async_copy(tbl_hbm.at[idx_v.at[q, 0]], rows_v.at[b],
                             gsem[b])

        def wait_gather(b):
            pltpu.make_async_copy(tbl_hbm.at[idx_v.at[0, 0]], rows_v.at[b],
                                  gsem[b]).wait()

        def issue_scatter(q, b):
            pltpu.async_copy(rows_v.at[b], acc.at[idx_v.at[q, 1]], ssem[b],
                             add=True)

        def wait_scatter(b):
            pltpu.make_async_copy(rows_v.at[b], acc.at[idx_v.at[0, 1]],
                                  ssem[b]).wait()

        for q in range(8):
            issue_idx(q, q)
        _zero_fill(zero_hbm, acc, s)
        plsc.subcore_barrier()
        for b in range(4):
            wait_idx(b)
            issue_gather(b, b)

        def body(g, carry):
            j0 = g * 8
            for u in range(8):
                j = j0 + u
                b = u % 4
                wait_gather(b)
                issue_scatter(u, b)
                wait_scatter(b)
                issue_idx(j + 8, u)
                wait_idx((u + 4) % 8)
                issue_gather((u + 4) % 8, b)
            return carry

        lax.fori_loop(0, cpt // 8 - 1, body, 0)
        for u in range(8):
            b = u % 4
            wait_gather(b)
            issue_scatter(u, b)
            wait_scatter(b)
            if u < 4:
                wait_idx((u + 4) % 8)
                issue_gather((u + 4) % 8, b)
        plsc.subcore_barrier()
        _writeback(acc, out_hbm, c, s)

    return prop


_sc_prop1 = _make_prop(split_edges=False)
_sc_prop2 = _make_prop(split_edges=True)


def _l1_body(x_ref, w1_ref, degp_ref, hp_ref, dinv_ref):
    deg = degp_ref[0, :, 0] + degp_ref[1, :, 0] + 1.0
    dinv = (1.0 / jnp.sqrt(deg))[:, None]
    dinv_ref[...] = dinv
    h = jnp.dot(x_ref[...], w1_ref[...], preferred_element_type=jnp.float32)
    hp = dinv * h
    hp_ref[0] = hp[:, :HALF]
    hp_ref[1] = hp[:, HALF:]


def _tc_l1(x, w1, degp):
    return pl.pallas_call(
        _l1_body,
        grid=(GRID,),
        in_specs=[
            pl.BlockSpec((BLK, D), lambda i: (i, 0)),
            pl.BlockSpec((D, H), lambda i: (0, 0)),
            pl.BlockSpec((NC, BLK, 128), lambda i: (0, i, 0)),
        ],
        out_specs=[
            pl.BlockSpec((NC, BLK, HALF), lambda i: (0, i, 0)),
            pl.BlockSpec((BLK, 1), lambda i: (i, 0)),
        ],
        out_shape=[
            jax.ShapeDtypeStruct((NC, N, HALF), jnp.float32),
            jax.ShapeDtypeStruct((N, 1), jnp.float32),
        ],
    )(x, w1, degp)


def _l2_body(acc_ref, hp_ref, dinv_ref, b1_ref, w2_ref, h2p_ref):
    a = jnp.concatenate([acc_ref[0], acc_ref[1]], axis=-1)
    hp = jnp.concatenate([hp_ref[0], hp_ref[1]], axis=-1)
    dinv = dinv_ref[...]
    h1 = jnp.maximum(dinv * (a + hp) + b1_ref[...], 0.0)
    h2p_ref[...] = dinv * jnp.dot(h1, w2_ref[...],
                                  preferred_element_type=jnp.float32)


def _tc_l2(acc1, hp, dinv, b1, w2p):
    return pl.pallas_call(
        _l2_body,
        grid=(GRID,),
        in_specs=[
            pl.BlockSpec((NC, BLK, HALF), lambda i: (0, i, 0)),
            pl.BlockSpec((NC, BLK, HALF), lambda i: (0, i, 0)),
            pl.BlockSpec((BLK, 1), lambda i: (i, 0)),
            pl.BlockSpec((1, H), lambda i: (0, 0)),
            pl.BlockSpec((H, C_PAD), lambda i: (0, 0)),
        ],
        out_specs=pl.BlockSpec((BLK, C_PAD), lambda i: (i, 0)),
        out_shape=jax.ShapeDtypeStruct((N, C_PAD), jnp.float32),
    )(acc1, hp, dinv, b1, w2p)


def _final_body(acc_ref, h2p_ref, dinv_ref, b2_ref, out_ref):
    a = acc_ref[0] + acc_ref[1] + h2p_ref[...]
    o = (dinv_ref[...] * a)[:, :C] + b2_ref[...]
    m = jnp.max(o, axis=1, keepdims=True)
    z = o - m
    lse = jnp.log(jnp.sum(jnp.exp(z), axis=1, keepdims=True))
    out_ref[...] = z - lse


def _tc_final(acc2, h2p, dinv, b2):
    return pl.pallas_call(
        _final_body,
        grid=(GRID,),
        in_specs=[
            pl.BlockSpec((NC, BLK, C_PAD), lambda i: (0, i, 0)),
            pl.BlockSpec((BLK, C_PAD), lambda i: (i, 0)),
            pl.BlockSpec((BLK, 1), lambda i: (i, 0)),
            pl.BlockSpec((1, C), lambda i: (0, 0)),
        ],
        out_specs=pl.BlockSpec((BLK, C), lambda i: (i, 0)),
        out_shape=jax.ShapeDtypeStruct((N, C), jnp.float32),
    )(acc2, h2p, dinv, b2)


@jax.jit
def kernel(x, edge_index, W1, b1, W2, b2):
    npad = E_PAD - E
    srcp = jnp.concatenate(
        [edge_index[0],
         jnp.arange(npad, dtype=jnp.int32) % 128]).reshape(NCHUNK, CHUNK)
    dstp = jnp.concatenate(
        [edge_index[1],
         N + (jnp.arange(npad, dtype=jnp.int32) % 128)]).reshape(NCHUNK, CHUNK)

    ones128 = jnp.ones((CHUNK, 128), jnp.float32)
    zero128 = jnp.zeros((RL, 128), jnp.float32)
    w2p = jnp.zeros((H, C_PAD), jnp.float32).at[:, :C].set(W2)

    degp = _sc_degree(dstp, ones128, zero128)
    hp, dinv = _tc_l1(x, W1, degp)
    idxp = jnp.stack([srcp, dstp], axis=1)
    acc1 = _sc_prop1(hp.reshape(NC * N, HALF), idxp, zero128)
    h2p = _tc_l2(acc1, hp, dinv, b1.reshape(1, H), w2p)
    acc2 = _sc_prop2(h2p, idxp, zero128)
    return _tc_final(acc2, h2p, dinv, b2.reshape(1, C))
